# baseline (speedup 1.0000x reference)
import functools

import jax
import jax.numpy as jnp
from jax import lax
from jax.experimental import pallas as pl
from jax.experimental.pallas import tpu as pltpu
from jax.experimental.pallas import tpu_sc as plsc

C = 192
GROUPS = 12
CLIP = 2
H = 64
W = 64
HW = H * W
ATTN = 9
CG = C // GROUPS
NA = CLIP * ATTN
NJ = NA
NJP = 24
NWT = 4 * NA
SB = HW // 128
PX = 64
NSUP = GROUPS * (HW // PX)
NW = 32
SPW = NSUP // NW
AB = 3
JB = 2 * AB
NB = NA // AB
SCALE = float(CG) ** -0.5
NEG = -3.0e38


def _proj_body(qf_ref, kf_ref, vf_ref, wq_ref, bq_ref, wk_ref, bk_ref,
               wv_ref, bv_ref, qt_ref, pw_ref):
    t = pl.program_id(0)

    @pl.when(t == 0)
    def _():
        qt_ref[...] = (jnp.dot(wq_ref[...], qf_ref[...],
                               preferred_element_type=jnp.float32)
                       + bq_ref[...][:, None]) * SCALE

    kp = (jnp.dot(wk_ref[...], kf_ref[0],
                  preferred_element_type=jnp.float32)
          + bk_ref[...][:, None])
    vp = (jnp.dot(wv_ref[...], vf_ref[0],
                  preferred_element_type=jnp.float32)
          + bv_ref[...][:, None])
    kb = lax.bitcast_convert_type(kp.astype(jnp.bfloat16), jnp.uint16)
    vb = lax.bitcast_convert_type(vp.astype(jnp.bfloat16), jnp.uint16)
    pw_ref[0] = (vb.astype(jnp.int32) << 16) | kb.astype(jnp.int32)


def _run_proj(qf, kf, vf, Wq, bq, Wk, bk, Wv, bv):
    full = lambda *s: pl.BlockSpec(s, lambda t: (0,) * len(s))
    return pl.pallas_call(
        _proj_body,
        grid=(CLIP,),
        in_specs=[
            full(C, HW),
            pl.BlockSpec((1, C, HW), lambda t: (t, 0, 0)),
            pl.BlockSpec((1, C, HW), lambda t: (t, 0, 0)),
            full(C, C), full(C), full(C, C), full(C), full(C, C), full(C),
        ],
        out_specs=(
            full(C, HW),
            pl.BlockSpec((1, C, HW), lambda t: (t, 0, 0)),
        ),
        out_shape=(
            jax.ShapeDtypeStruct((C, HW), jnp.float32),
            jax.ShapeDtypeStruct((CLIP, C, HW), jnp.int32),
        ),
    )(qf, kf, vf, Wq, bq, Wk, bk, Wv, bv)


def _gen_body(off_ref, idx_ref, wgt_ref):
    g = pl.program_id(0)
    shp = (SB, 128)
    i0 = lax.broadcasted_iota(jnp.int32, shp, 0)
    i1 = lax.broadcasted_iota(jnp.int32, shp, 1)
    yi = 2 * i0 + (i1 >> 6)
    xi = i1 & 63
    yf = yi.astype(jnp.float32)
    xf = xi.astype(jnp.float32)
    for t in range(CLIP):
        base = (t * GROUPS + g) * HW
        for a in range(ATTN):
            dy = float(a // 3 - 1)
            dx = float(a % 3 - 1)
            sy = yf + dy + off_ref[t, 0, a, 0]
            sx = xf + dx + off_ref[t, 0, a, 1]
            y0 = jnp.floor(sy)
            x0 = jnp.floor(sx)
            wy = sy - y0
            wx = sx - x0
            vy0 = ((y0 >= 0.0) & (y0 <= H - 1.0)).astype(jnp.float32)
            vy1 = ((y0 >= -1.0) & (y0 <= H - 2.0)).astype(jnp.float32)
            vx0 = ((x0 >= 0.0) & (x0 <= W - 1.0)).astype(jnp.float32)
            vx1 = ((x0 >= -1.0) & (x0 <= W - 2.0)).astype(jnp.float32)
            yc0 = jnp.clip(y0, 0.0, H - 1.0).astype(jnp.int32)
            xc0 = jnp.clip(x0, 0.0, W - 1.0).astype(jnp.int32)
            negx = x0 < 0.0
            wh0 = jnp.where(negx, wx * vx1, (1.0 - wx) * vx0)
            wh1 = jnp.where(negx, 0.0, wx * vx1)
            negy = y0 < 0.0
            wv0 = jnp.where(negy, wy * vy1, (1.0 - wy) * vy0)
            wv1 = jnp.where(negy, 0.0, wy * vy1)
            aa = t * ATTN + a
            idx_ref[0, :, aa, :] = base + yc0 * W + xc0
            wgt_ref[0, :, 4 * aa + 0, :] = wv0 * wh0
            wgt_ref[0, :, 4 * aa + 1, :] = wv0 * wh1
            wgt_ref[0, :, 4 * aa + 2, :] = wv1 * wh0
            wgt_ref[0, :, 4 * aa + 3, :] = wv1 * wh1


def _run_gen(offr):
    return pl.pallas_call(
        _gen_body,
        grid=(GROUPS,),
        in_specs=[pl.BlockSpec((CLIP, 1, ATTN, 2, SB, 128),
                               lambda g: (0, g, 0, 0, 0, 0))],
        out_specs=(
            pl.BlockSpec((1, SB, NJP, 128), lambda g: (g, 0, 0, 0)),
            pl.BlockSpec((1, SB, NWT, 128), lambda g: (g, 0, 0, 0)),
        ),
        out_shape=(
            jax.ShapeDtypeStruct((GROUPS, SB, NJP, 128), jnp.int32),
            jax.ShapeDtypeStruct((GROUPS, SB, NWT, 128), jnp.float32),
        ),
    )(offr)


def _sc_attn(tab, idx, wgt, qt):
    mesh = plsc.VectorSubcoreMesh(core_axis_name="c", subcore_axis_name="s")
    hmask = jnp.int32(-65536)

    @functools.partial(
        pl.kernel,
        out_type=jax.ShapeDtypeStruct((C, HW), jnp.float32),
        mesh=mesh,
        compiler_params=pltpu.CompilerParams(use_tc_tiling_on_sc=False,
                                             needs_layout_passes=False),
        scratch_types=[
            pltpu.VMEM((2, NJ, PX), jnp.int32),
            pltpu.VMEM((2, NWT, PX), jnp.float32),
            pltpu.VMEM((2, CG, PX), jnp.float32),
            pltpu.VMEM((NJ, PX, 4 * CG), jnp.int32),
            pltpu.VMEM((2, PX), jnp.float32),
            pltpu.VMEM((CG, PX), jnp.float32),
            pltpu.SemaphoreType.DMA,
            pltpu.SemaphoreType.DMA,
            pltpu.SemaphoreType.DMA,
            pltpu.SemaphoreType.DMA,
            pltpu.SemaphoreType.DMA,
            pltpu.SemaphoreType.DMA,
            pltpu.SemaphoreType.DMA,
        ],
    )
    def run(tab_hbm, idx_hbm, wgt_hbm, qt_hbm, out_hbm,
            idx_v, wgt_v, q_v, rows_v, mden_v, out_v,
            gsem0, gsem1, gsem2, gsem3, gsem4, gsem5, msem):
        wid = lax.axis_index("s") * 2 + lax.axis_index("c")
        iota = lax.iota(jnp.int32, 16)
        gsems = (gsem0, gsem1, gsem2, gsem3, gsem4, gsem5)
        nlg = PX // 16
        spg = HW // PX

        def meta_srcs(sup):
            g = sup // spg
            sb = (sup % spg) // 2
            hf = sup % 2
            px0 = (sup % spg) * PX
            return (idx_hbm.at[g, sb, pl.ds(0, NJ), pl.ds(hf * PX, PX)],
                    wgt_hbm.at[g, sb, :, pl.ds(hf * PX, PX)],
                    qt_hbm.at[pl.ds(g * CG, CG), pl.ds(px0, PX)])

        i0src, w0src, q0src = meta_srcs(wid * SPW)
        pltpu.sync_copy(i0src, idx_v.at[0])
        pltpu.sync_copy(w0src, wgt_v.at[0])
        pltpu.sync_copy(q0src, q_v.at[0])

        def sup_body(i, _):
            sup = wid * SPW + i
            g = sup // spg
            px0 = (sup % spg) * PX
            ps = lax.rem(i, 2)

            def issue(b):
                return [
                    pltpu.async_copy(tab_hbm.at[idx_v.at[ps, b * AB + jl]],
                                     rows_v.at[b * AB + jl],
                                     gsems[b])
                    for jl in range(AB)
                ]

            def init_body(lg, _):
                sl = pl.ds(lg * 16, 16)
                mden_v[0, sl] = jnp.full((16,), NEG, jnp.float32)
                mden_v[1, sl] = jnp.zeros((16,), jnp.float32)
                for c in range(CG):
                    out_v[c, sl] = jnp.zeros((16,), jnp.float32)
                return 0

            lax.fori_loop(0, nlg, init_body, 0)

            def compute(b):
                def a_body(al, _):
                    aa = b * AB + al

                    def lg_body(lg, _):
                        sl = pl.ds(lg * 16, 16)
                        lanes = lg * 16 + iota
                        ws = [wgt_v[ps, aa * 4 + tt, sl]
                              for tt in range(4)]
                        jv = jnp.full((16,), b * AB + al, jnp.int32)
                        s = jnp.zeros((16,), jnp.float32)
                        vss = []
                        for c in range(CG):
                            ks = jnp.zeros((16,), jnp.float32)
                            vs = jnp.zeros((16,), jnp.float32)
                            for tt in range(4):
                                cc = jnp.full((16,), tt * CG + c,
                                              jnp.int32)
                                word = plsc.load_gather(
                                    rows_v, [jv, lanes, cc])
                                kf = plsc.bitcast(word << 16,
                                                  jnp.float32)
                                vf = plsc.bitcast(word & hmask,
                                                  jnp.float32)
                                ks = ks + ws[tt] * kf
                                vs = vs + ws[tt] * vf
                            s = s + q_v[ps, c, sl] * ks
                            vss.append(vs)
                        m0 = mden_v[0, sl]
                        m1 = jnp.maximum(m0, s)
                        c1 = jnp.exp(m0 - m1)
                        e = jnp.exp(s - m1)
                        mden_v[0, sl] = m1
                        mden_v[1, sl] = mden_v[1, sl] * c1 + e
                        for c in range(CG):
                            out_v[c, sl] = out_v[c, sl] * c1 + e * vss[c]
                        return 0

                    lax.fori_loop(0, nlg, lg_body, 0)
                    return 0

                lax.fori_loop(0, AB, a_body, 0)

            with jax.named_scope("gissue"):
                pends = [issue(b) for b in range(NB)]
            for b in range(NB):
                with jax.named_scope("gwait"):
                    for cp in pends[b]:
                        cp.wait()
                if b == 0:
                    @pl.when(i < SPW - 1)
                    def _():
                        isrc, wsrc, qsrc = meta_srcs(sup + 1)
                        pn = 1 - ps
                        pltpu.async_copy(isrc, idx_v.at[pn], msem)
                        pltpu.async_copy(wsrc, wgt_v.at[pn], msem)
                        pltpu.async_copy(qsrc, q_v.at[pn], msem)
                with jax.named_scope("cmp"):
                    compute(b)

            def norm_body(lg, _):
                sl = pl.ds(lg * 16, 16)
                inv = 1.0 / mden_v[1, sl]
                for c in range(CG):
                    out_v[c, sl] = out_v[c, sl] * inv
                return 0

            lax.fori_loop(0, nlg, norm_body, 0)
            with jax.named_scope("outw"):
                pltpu.sync_copy(
                    out_v, out_hbm.at[pl.ds(g * CG, CG), pl.ds(px0, PX)])

            @pl.when(i < SPW - 1)
            def _():
                with jax.named_scope("mwait"):
                    isrc, wsrc, qsrc = meta_srcs(sup + 1)
                    pn = 1 - ps
                    pltpu.make_async_copy(isrc, idx_v.at[pn], msem).wait()
                    pltpu.make_async_copy(wsrc, wgt_v.at[pn], msem).wait()
                    pltpu.make_async_copy(qsrc, q_v.at[pn], msem).wait()

            return 0

        lax.fori_loop(0, SPW, sup_body, 0)

    return run(tab, idx, wgt, qt)


def _mlp_body(x_ref, w1_ref, b1_ref, w2_ref, b2_ref, o_ref):
    h = (jnp.dot(w1_ref[...], x_ref[...],
                 preferred_element_type=jnp.float32)
         + b1_ref[...][:, None])
    h = h * 0.5 * (1.0 + lax.erf(h * (2.0 ** -0.5)))
    o_ref[...] = (x_ref[...]
                  + jnp.dot(w2_ref[...], h,
                            preferred_element_type=jnp.float32)
                  + b2_ref[...][:, None])


def _run_mlp(x, W1, b1, W2, b2):
    return pl.pallas_call(
        _mlp_body,
        out_shape=jax.ShapeDtypeStruct((C, HW), jnp.float32),
    )(x, W1, b1, W2, b2)


def kernel(q, k, v, offset, Wq, bq, Wk, bk, Wv, bv, W1, b1, W2, b2):
    qf = q.reshape(C, HW)
    kf = k.reshape(CLIP, C, HW)
    vf = v.reshape(CLIP, C, HW)
    offr = offset.reshape(CLIP, GROUPS, ATTN, 2, SB, 128)
    qt, pw = _run_proj(qf, kf, vf, Wq, bq, Wk, bk, Wv, bv)
    wpm = pw.reshape(CLIP, GROUPS, CG, HW).transpose(0, 1, 3, 2)
    tab = jnp.concatenate(
        [wpm, jnp.roll(wpm, -1, axis=2), jnp.roll(wpm, -W, axis=2),
         jnp.roll(wpm, -W - 1, axis=2)],
        axis=3).reshape(CLIP * GROUPS * HW, 4 * CG)
    idx, wgt = _run_gen(offr)
    attn_out = _sc_attn(tab, idx, wgt, qt)
    out = _run_mlp(attn_out, W1, b1, W2, b2)
    return out.reshape(1, 1, C, H, W)


# speedup vs baseline: 1.6287x; 1.6287x over previous
import functools

import jax
import jax.numpy as jnp
from jax import lax
from jax.experimental import pallas as pl
from jax.experimental.pallas import tpu as pltpu
from jax.experimental.pallas import tpu_sc as plsc

C = 192
GROUPS = 12
CLIP = 2
H = 64
W = 64
HW = H * W
ATTN = 9
CG = C // GROUPS
NA = CLIP * ATTN
NJ = NA
NJP = 24
NWT = 4 * NA
SB = HW // 128
PX = 64
NSUP = GROUPS * (HW // PX)
NW = 32
SPW = NSUP // NW
AB = 3
JB = 2 * AB
NB = NA // AB
SCALE = float(CG) ** -0.5
NEG = -3.0e38


def _proj_body(qf_ref, kf_ref, vf_ref, wq_ref, bq_ref, wk_ref, bk_ref,
               wv_ref, bv_ref, qt_ref, pw_ref):
    t = pl.program_id(0)

    @pl.when(t == 0)
    def _():
        qt_ref[...] = (jnp.dot(wq_ref[...], qf_ref[...],
                               preferred_element_type=jnp.float32)
                       + bq_ref[...][:, None]) * SCALE

    kp = (jnp.dot(wk_ref[...], kf_ref[0],
                  preferred_element_type=jnp.float32)
          + bk_ref[...][:, None])
    vp = (jnp.dot(wv_ref[...], vf_ref[0],
                  preferred_element_type=jnp.float32)
          + bv_ref[...][:, None])
    kb = lax.bitcast_convert_type(kp.astype(jnp.bfloat16), jnp.uint16)
    vb = lax.bitcast_convert_type(vp.astype(jnp.bfloat16), jnp.uint16)
    pw_ref[0] = (vb.astype(jnp.int32) << 16) | kb.astype(jnp.int32)


def _run_proj(qf, kf, vf, Wq, bq, Wk, bk, Wv, bv):
    full = lambda *s: pl.BlockSpec(s, lambda t: (0,) * len(s))
    return pl.pallas_call(
        _proj_body,
        grid=(CLIP,),
        in_specs=[
            full(C, HW),
            pl.BlockSpec((1, C, HW), lambda t: (t, 0, 0)),
            pl.BlockSpec((1, C, HW), lambda t: (t, 0, 0)),
            full(C, C), full(C), full(C, C), full(C), full(C, C), full(C),
        ],
        out_specs=(
            full(C, HW),
            pl.BlockSpec((1, C, HW), lambda t: (t, 0, 0)),
        ),
        out_shape=(
            jax.ShapeDtypeStruct((C, HW), jnp.float32),
            jax.ShapeDtypeStruct((CLIP, C, HW), jnp.int32),
        ),
    )(qf, kf, vf, Wq, bq, Wk, bk, Wv, bv)


def _gen_body(off_ref, idx_ref, wgt_ref):
    g = pl.program_id(0)
    shp = (SB, 128)
    i0 = lax.broadcasted_iota(jnp.int32, shp, 0)
    i1 = lax.broadcasted_iota(jnp.int32, shp, 1)
    yi = 2 * i0 + (i1 >> 6)
    xi = i1 & 63
    yf = yi.astype(jnp.float32)
    xf = xi.astype(jnp.float32)
    for t in range(CLIP):
        base = (t * GROUPS + g) * HW
        for a in range(ATTN):
            dy = float(a // 3 - 1)
            dx = float(a % 3 - 1)
            sy = yf + dy + off_ref[t, 0, a, 0]
            sx = xf + dx + off_ref[t, 0, a, 1]
            y0 = jnp.floor(sy)
            x0 = jnp.floor(sx)
            wy = sy - y0
            wx = sx - x0
            vy0 = ((y0 >= 0.0) & (y0 <= H - 1.0)).astype(jnp.float32)
            vy1 = ((y0 >= -1.0) & (y0 <= H - 2.0)).astype(jnp.float32)
            vx0 = ((x0 >= 0.0) & (x0 <= W - 1.0)).astype(jnp.float32)
            vx1 = ((x0 >= -1.0) & (x0 <= W - 2.0)).astype(jnp.float32)
            yc0 = jnp.clip(y0, 0.0, H - 1.0).astype(jnp.int32)
            xc0 = jnp.clip(x0, 0.0, W - 1.0).astype(jnp.int32)
            negx = x0 < 0.0
            wh0 = jnp.where(negx, wx * vx1, (1.0 - wx) * vx0)
            wh1 = jnp.where(negx, 0.0, wx * vx1)
            negy = y0 < 0.0
            wv0 = jnp.where(negy, wy * vy1, (1.0 - wy) * vy0)
            wv1 = jnp.where(negy, 0.0, wy * vy1)
            aa = t * ATTN + a
            idx_ref[0, :, aa, :] = base + yc0 * W + xc0
            wgt_ref[0, :, 4 * aa + 0, :] = wv0 * wh0
            wgt_ref[0, :, 4 * aa + 1, :] = wv0 * wh1
            wgt_ref[0, :, 4 * aa + 2, :] = wv1 * wh0
            wgt_ref[0, :, 4 * aa + 3, :] = wv1 * wh1


def _run_gen(offr):
    return pl.pallas_call(
        _gen_body,
        grid=(GROUPS,),
        in_specs=[pl.BlockSpec((CLIP, 1, ATTN, 2, SB, 128),
                               lambda g: (0, g, 0, 0, 0, 0))],
        out_specs=(
            pl.BlockSpec((1, SB, NJP, 128), lambda g: (g, 0, 0, 0)),
            pl.BlockSpec((1, SB, NWT, 128), lambda g: (g, 0, 0, 0)),
        ),
        out_shape=(
            jax.ShapeDtypeStruct((GROUPS, SB, NJP, 128), jnp.int32),
            jax.ShapeDtypeStruct((GROUPS, SB, NWT, 128), jnp.float32),
        ),
    )(offr)


def _sc_attn(tab, idx, wgt, qt):
    mesh = plsc.VectorSubcoreMesh(core_axis_name="c", subcore_axis_name="s")
    hmask = jnp.int32(-65536)

    @functools.partial(
        pl.kernel,
        out_type=jax.ShapeDtypeStruct((C, HW), jnp.float32),
        mesh=mesh,
        compiler_params=pltpu.CompilerParams(use_tc_tiling_on_sc=False,
                                             needs_layout_passes=False),
        scratch_types=[
            pltpu.VMEM((2, NJ, PX), jnp.int32),
            pltpu.VMEM((2, NWT, PX), jnp.float32),
            pltpu.VMEM((2, CG, PX), jnp.float32),
            pltpu.VMEM((NJ, PX, 4 * CG + 1), jnp.int32),
            pltpu.VMEM((2, PX), jnp.float32),
            pltpu.VMEM((CG, PX), jnp.float32),
            pltpu.SemaphoreType.DMA,
            pltpu.SemaphoreType.DMA,
            pltpu.SemaphoreType.DMA,
            pltpu.SemaphoreType.DMA,
            pltpu.SemaphoreType.DMA,
            pltpu.SemaphoreType.DMA,
            pltpu.SemaphoreType.DMA,
        ],
    )
    def run(tab_hbm, idx_hbm, wgt_hbm, qt_hbm, out_hbm,
            idx_v, wgt_v, q_v, rows_v, mden_v, out_v,
            gsem0, gsem1, gsem2, gsem3, gsem4, gsem5, msem):
        wid = lax.axis_index("s") * 2 + lax.axis_index("c")
        iota = lax.iota(jnp.int32, 16)
        gsems = (gsem0, gsem1, gsem2, gsem3, gsem4, gsem5)
        nlg = PX // 16
        spg = HW // PX

        def meta_srcs(sup):
            g = sup // spg
            sb = (sup % spg) // 2
            hf = sup % 2
            px0 = (sup % spg) * PX
            return (idx_hbm.at[g, sb, pl.ds(0, NJ), pl.ds(hf * PX, PX)],
                    wgt_hbm.at[g, sb, :, pl.ds(hf * PX, PX)],
                    qt_hbm.at[pl.ds(g * CG, CG), pl.ds(px0, PX)])

        i0src, w0src, q0src = meta_srcs(wid * SPW)
        pltpu.sync_copy(i0src, idx_v.at[0])
        pltpu.sync_copy(w0src, wgt_v.at[0])
        pltpu.sync_copy(q0src, q_v.at[0])

        def sup_body(i, _):
            sup = wid * SPW + i
            g = sup // spg
            px0 = (sup % spg) * PX
            ps = lax.rem(i, 2)

            def issue(b):
                return [
                    pltpu.async_copy(tab_hbm.at[idx_v.at[ps, b * AB + jl]],
                                     rows_v.at[b * AB + jl],
                                     gsems[b])
                    for jl in range(AB)
                ]

            def init_body(lg, _):
                sl = pl.ds(lg * 16, 16)
                mden_v[0, sl] = jnp.full((16,), NEG, jnp.float32)
                mden_v[1, sl] = jnp.zeros((16,), jnp.float32)
                for c in range(CG):
                    out_v[c, sl] = jnp.zeros((16,), jnp.float32)
                return 0

            lax.fori_loop(0, nlg, init_body, 0)

            def compute(b):
                def a_body(al, _):
                    aa = b * AB + al

                    def lg_body(lg, _):
                        sl = pl.ds(lg * 16, 16)
                        lanes = lg * 16 + iota
                        ws = [wgt_v[ps, aa * 4 + tt, sl]
                              for tt in range(4)]
                        jv = jnp.full((16,), b * AB + al, jnp.int32)
                        s = jnp.zeros((16,), jnp.float32)
                        vss = []
                        for c in range(CG):
                            ks = jnp.zeros((16,), jnp.float32)
                            vs = jnp.zeros((16,), jnp.float32)
                            for tt in range(4):
                                cc = jnp.full((16,), tt * CG + c,
                                              jnp.int32)
                                word = plsc.load_gather(
                                    rows_v, [jv, lanes, cc])
                                kf = plsc.bitcast(word << 16,
                                                  jnp.float32)
                                vf = plsc.bitcast(word & hmask,
                                                  jnp.float32)
                                ks = ks + ws[tt] * kf
                                vs = vs + ws[tt] * vf
                            s = s + q_v[ps, c, sl] * ks
                            vss.append(vs)
                        m0 = mden_v[0, sl]
                        m1 = jnp.maximum(m0, s)
                        c1 = jnp.exp(m0 - m1)
                        e = jnp.exp(s - m1)
                        mden_v[0, sl] = m1
                        mden_v[1, sl] = mden_v[1, sl] * c1 + e
                        for c in range(CG):
                            out_v[c, sl] = out_v[c, sl] * c1 + e * vss[c]
                        return 0

                    lax.fori_loop(0, nlg, lg_body, 0)
                    return 0

                lax.fori_loop(0, AB, a_body, 0)

            with jax.named_scope("gissue"):
                pends = [issue(b) for b in range(NB)]
            for b in range(NB):
                with jax.named_scope("gwait"):
                    for cp in pends[b]:
                        cp.wait()
                if b == 0:
                    @pl.when(i < SPW - 1)
                    def _():
                        isrc, wsrc, qsrc = meta_srcs(sup + 1)
                        pn = 1 - ps
                        pltpu.async_copy(isrc, idx_v.at[pn], msem)
                        pltpu.async_copy(wsrc, wgt_v.at[pn], msem)
                        pltpu.async_copy(qsrc, q_v.at[pn], msem)
                with jax.named_scope("cmp"):
                    compute(b)

            def norm_body(lg, _):
                sl = pl.ds(lg * 16, 16)
                inv = 1.0 / mden_v[1, sl]
                for c in range(CG):
                    out_v[c, sl] = out_v[c, sl] * inv
                return 0

            lax.fori_loop(0, nlg, norm_body, 0)
            with jax.named_scope("outw"):
                pltpu.sync_copy(
                    out_v, out_hbm.at[pl.ds(g * CG, CG), pl.ds(px0, PX)])

            @pl.when(i < SPW - 1)
            def _():
                with jax.named_scope("mwait"):
                    isrc, wsrc, qsrc = meta_srcs(sup + 1)
                    pn = 1 - ps
                    pltpu.make_async_copy(isrc, idx_v.at[pn], msem).wait()
                    pltpu.make_async_copy(wsrc, wgt_v.at[pn], msem).wait()
                    pltpu.make_async_copy(qsrc, q_v.at[pn], msem).wait()

            return 0

        lax.fori_loop(0, SPW, sup_body, 0)

    return run(tab, idx, wgt, qt)


def _mlp_body(x_ref, w1_ref, b1_ref, w2_ref, b2_ref, o_ref):
    h = (jnp.dot(w1_ref[...], x_ref[...],
                 preferred_element_type=jnp.float32)
         + b1_ref[...][:, None])
    h = h * 0.5 * (1.0 + lax.erf(h * (2.0 ** -0.5)))
    o_ref[...] = (x_ref[...]
                  + jnp.dot(w2_ref[...], h,
                            preferred_element_type=jnp.float32)
                  + b2_ref[...][:, None])


def _run_mlp(x, W1, b1, W2, b2):
    return pl.pallas_call(
        _mlp_body,
        out_shape=jax.ShapeDtypeStruct((C, HW), jnp.float32),
    )(x, W1, b1, W2, b2)


def kernel(q, k, v, offset, Wq, bq, Wk, bk, Wv, bv, W1, b1, W2, b2):
    qf = q.reshape(C, HW)
    kf = k.reshape(CLIP, C, HW)
    vf = v.reshape(CLIP, C, HW)
    offr = offset.reshape(CLIP, GROUPS, ATTN, 2, SB, 128)
    qt, pw = _run_proj(qf, kf, vf, Wq, bq, Wk, bk, Wv, bv)
    wpm = pw.reshape(CLIP, GROUPS, CG, HW).transpose(0, 1, 3, 2)
    tab = jnp.concatenate(
        [wpm, jnp.roll(wpm, -1, axis=2), jnp.roll(wpm, -W, axis=2),
         jnp.roll(wpm, -W - 1, axis=2),
         jnp.zeros((CLIP, GROUPS, HW, 1), jnp.int32)],
        axis=3).reshape(CLIP * GROUPS * HW, 4 * CG + 1)
    idx, wgt = _run_gen(offr)
    attn_out = _sc_attn(tab, idx, wgt, qt)
    out = _run_mlp(attn_out, W1, b1, W2, b2)
    return out.reshape(1, 1, C, H, W)


# speedup vs baseline: 1.9504x; 1.1975x over previous
import functools

import jax
import jax.numpy as jnp
from jax import lax
from jax.experimental import pallas as pl
from jax.experimental.pallas import tpu as pltpu
from jax.experimental.pallas import tpu_sc as plsc

C = 192
GROUPS = 12
CLIP = 2
H = 64
W = 64
HW = H * W
ATTN = 9
CG = C // GROUPS
NA = CLIP * ATTN
NJ = NA
NJP = 24
NWT = 4 * NA
SB = HW // 128
PX = 64
NSUP = GROUPS * (HW // PX)
NW = 32
SPW = NSUP // NW
AB = 3
JB = 2 * AB
NB = NA // AB
SCALE = float(CG) ** -0.5
NEG = -3.0e38


def _proj_body(qf_ref, kf_ref, vf_ref, wq_ref, bq_ref, wk_ref, bk_ref,
               wv_ref, bv_ref, qt_ref, pw_ref):
    t = pl.program_id(0)

    @pl.when(t == 0)
    def _():
        qt_ref[...] = (jnp.dot(wq_ref[...], qf_ref[...],
                               preferred_element_type=jnp.float32)
                       + bq_ref[...][:, None]) * SCALE

    kp = (jnp.dot(wk_ref[...], kf_ref[0],
                  preferred_element_type=jnp.float32)
          + bk_ref[...][:, None])
    vp = (jnp.dot(wv_ref[...], vf_ref[0],
                  preferred_element_type=jnp.float32)
          + bv_ref[...][:, None])
    kb = lax.bitcast_convert_type(kp.astype(jnp.bfloat16), jnp.uint16)
    vb = lax.bitcast_convert_type(vp.astype(jnp.bfloat16), jnp.uint16)
    pw_ref[0] = (vb.astype(jnp.int32) << 16) | kb.astype(jnp.int32)


def _run_proj(qf, kf, vf, Wq, bq, Wk, bk, Wv, bv):
    full = lambda *s: pl.BlockSpec(s, lambda t: (0,) * len(s))
    return pl.pallas_call(
        _proj_body,
        grid=(CLIP,),
        in_specs=[
            full(C, HW),
            pl.BlockSpec((1, C, HW), lambda t: (t, 0, 0)),
            pl.BlockSpec((1, C, HW), lambda t: (t, 0, 0)),
            full(C, C), full(C), full(C, C), full(C), full(C, C), full(C),
        ],
        out_specs=(
            full(C, HW),
            pl.BlockSpec((1, C, HW), lambda t: (t, 0, 0)),
        ),
        out_shape=(
            jax.ShapeDtypeStruct((C, HW), jnp.float32),
            jax.ShapeDtypeStruct((CLIP, C, HW), jnp.int32),
        ),
    )(qf, kf, vf, Wq, bq, Wk, bk, Wv, bv)


def _gen_body(off_ref, idx_ref, wgt_ref):
    g = pl.program_id(0)
    shp = (SB, 128)
    i0 = lax.broadcasted_iota(jnp.int32, shp, 0)
    i1 = lax.broadcasted_iota(jnp.int32, shp, 1)
    yi = 2 * i0 + (i1 >> 6)
    xi = i1 & 63
    yf = yi.astype(jnp.float32)
    xf = xi.astype(jnp.float32)
    for t in range(CLIP):
        base = (t * GROUPS + g) * HW
        for a in range(ATTN):
            dy = float(a // 3 - 1)
            dx = float(a % 3 - 1)
            sy = yf + dy + off_ref[t, 0, a, 0]
            sx = xf + dx + off_ref[t, 0, a, 1]
            y0 = jnp.floor(sy)
            x0 = jnp.floor(sx)
            wy = sy - y0
            wx = sx - x0
            vy0 = ((y0 >= 0.0) & (y0 <= H - 1.0)).astype(jnp.float32)
            vy1 = ((y0 >= -1.0) & (y0 <= H - 2.0)).astype(jnp.float32)
            vx0 = ((x0 >= 0.0) & (x0 <= W - 1.0)).astype(jnp.float32)
            vx1 = ((x0 >= -1.0) & (x0 <= W - 2.0)).astype(jnp.float32)
            yc0 = jnp.clip(y0, 0.0, H - 1.0).astype(jnp.int32)
            xc0 = jnp.clip(x0, 0.0, W - 1.0).astype(jnp.int32)
            negx = x0 < 0.0
            wh0 = jnp.where(negx, wx * vx1, (1.0 - wx) * vx0)
            wh1 = jnp.where(negx, 0.0, wx * vx1)
            negy = y0 < 0.0
            wv0 = jnp.where(negy, wy * vy1, (1.0 - wy) * vy0)
            wv1 = jnp.where(negy, 0.0, wy * vy1)
            aa = t * ATTN + a
            idx_ref[0, :, aa, :] = base + yc0 * W + xc0
            wgt_ref[0, :, 4 * aa + 0, :] = wv0 * wh0
            wgt_ref[0, :, 4 * aa + 1, :] = wv0 * wh1
            wgt_ref[0, :, 4 * aa + 2, :] = wv1 * wh0
            wgt_ref[0, :, 4 * aa + 3, :] = wv1 * wh1


def _run_gen(offr):
    return pl.pallas_call(
        _gen_body,
        grid=(GROUPS,),
        in_specs=[pl.BlockSpec((CLIP, 1, ATTN, 2, SB, 128),
                               lambda g: (0, g, 0, 0, 0, 0))],
        out_specs=(
            pl.BlockSpec((1, SB, NJP, 128), lambda g: (g, 0, 0, 0)),
            pl.BlockSpec((1, SB, NWT, 128), lambda g: (g, 0, 0, 0)),
        ),
        out_shape=(
            jax.ShapeDtypeStruct((GROUPS, SB, NJP, 128), jnp.int32),
            jax.ShapeDtypeStruct((GROUPS, SB, NWT, 128), jnp.float32),
        ),
    )(offr)


def _sc_attn(tab, idx, wgt, qt):
    mesh = plsc.VectorSubcoreMesh(core_axis_name="c", subcore_axis_name="s")
    hmask = jnp.int32(-65536)

    @functools.partial(
        pl.kernel,
        out_type=jax.ShapeDtypeStruct((C, HW), jnp.float32),
        mesh=mesh,
        compiler_params=pltpu.CompilerParams(use_tc_tiling_on_sc=False,
                                             needs_layout_passes=False),
        scratch_types=[
            pltpu.VMEM((2, NJ, PX), jnp.int32),
            pltpu.VMEM((2, NWT, PX), jnp.float32),
            pltpu.VMEM((2, CG, PX), jnp.float32),
            pltpu.VMEM((NJ, PX, 4 * CG), jnp.int32),
            pltpu.VMEM((2, PX), jnp.float32),
            pltpu.VMEM((CG, PX), jnp.float32),
            pltpu.VMEM((CG, PX), jnp.float32),
            pltpu.SemaphoreType.DMA,
            pltpu.SemaphoreType.DMA,
            pltpu.SemaphoreType.DMA,
            pltpu.SemaphoreType.DMA,
            pltpu.SemaphoreType.DMA,
            pltpu.SemaphoreType.DMA,
            pltpu.SemaphoreType.DMA,
        ],
    )
    def run(tab_hbm, idx_hbm, wgt_hbm, qt_hbm, out_hbm,
            idx_v, wgt_v, q_v, rows_v, mden_v, out_v, out2_v,
            gsem0, gsem1, gsem2, gsem3, gsem4, gsem5, msem):
        wid = lax.axis_index("s") * 2 + lax.axis_index("c")
        iota = lax.iota(jnp.int32, 16)
        gsems = (gsem0, gsem1, gsem2, gsem3, gsem4, gsem5)
        nlg = PX // 16
        spg = HW // PX

        def meta_srcs(sup):
            g = sup // spg
            sb = (sup % spg) // 2
            hf = sup % 2
            px0 = (sup % spg) * PX
            return (idx_hbm.at[g, sb, pl.ds(0, NJ), pl.ds(hf * PX, PX)],
                    wgt_hbm.at[g, sb, :, pl.ds(hf * PX, PX)],
                    qt_hbm.at[pl.ds(g * CG, CG), pl.ds(px0, PX)])

        i0src, w0src, q0src = meta_srcs(wid * SPW)
        pltpu.sync_copy(i0src, idx_v.at[0])
        pltpu.sync_copy(w0src, wgt_v.at[0])
        pltpu.sync_copy(q0src, q_v.at[0])

        def sup_body(i, _):
            sup = wid * SPW + i
            g = sup // spg
            px0 = (sup % spg) * PX
            ps = lax.rem(i, 2)

            def issue(b):
                return [
                    pltpu.async_copy(tab_hbm.at[idx_v.at[ps, b * AB + jl]],
                                     rows_v.at[b * AB + jl],
                                     gsems[b])
                    for jl in range(AB)
                ]

            def init_body(lg, _):
                sl = pl.ds(lg * 16, 16)
                mden_v[0, sl] = jnp.full((16,), NEG, jnp.float32)
                mden_v[1, sl] = jnp.zeros((16,), jnp.float32)
                for c in range(CG):
                    out_v[c, sl] = jnp.zeros((16,), jnp.float32)
                return 0

            lax.fori_loop(0, nlg, init_body, 0)

            def compute(b):
                def a_body(al, _):
                    aa = b * AB + al

                    def lg_body(lg, _):
                        sl = pl.ds(lg * 16, 16)
                        lanes = lg * 16 + iota
                        ws = [wgt_v[ps, aa * 4 + tt, sl]
                              for tt in range(4)]
                        jv = jnp.full((16,), b * AB + al, jnp.int32)
                        psv = jnp.full((16,), ps, jnp.int32)
                        s = jnp.zeros((16,), jnp.float32)
                        vss = []
                        for c in range(CG):
                            crot = (jnp.full((16,), c, jnp.int32)
                                    + iota) & (CG - 1)
                            ks = jnp.zeros((16,), jnp.float32)
                            vs = jnp.zeros((16,), jnp.float32)
                            for tt in range(4):
                                word = plsc.load_gather(
                                    rows_v, [jv, lanes, crot + tt * CG])
                                kf = plsc.bitcast(word << 16,
                                                  jnp.float32)
                                vf = plsc.bitcast(word & hmask,
                                                  jnp.float32)
                                ks = ks + ws[tt] * kf
                                vs = vs + ws[tt] * vf
                            qv = plsc.load_gather(q_v, [psv, crot, lanes])
                            s = s + qv * ks
                            vss.append(vs)
                        m0 = mden_v[0, sl]
                        m1 = jnp.maximum(m0, s)
                        c1 = jnp.exp(m0 - m1)
                        e = jnp.exp(s - m1)
                        mden_v[0, sl] = m1
                        mden_v[1, sl] = mden_v[1, sl] * c1 + e
                        for c in range(CG):
                            out_v[c, sl] = out_v[c, sl] * c1 + e * vss[c]
                        return 0

                    lax.fori_loop(0, nlg, lg_body, 0)
                    return 0

                lax.fori_loop(0, AB, a_body, 0)

            with jax.named_scope("gissue"):
                pends = [issue(b) for b in range(NB)]
            for b in range(NB):
                with jax.named_scope("gwait"):
                    for cp in pends[b]:
                        cp.wait()
                if b == 0:
                    @pl.when(i < SPW - 1)
                    def _():
                        isrc, wsrc, qsrc = meta_srcs(sup + 1)
                        pn = 1 - ps
                        pltpu.async_copy(isrc, idx_v.at[pn], msem)
                        pltpu.async_copy(wsrc, wgt_v.at[pn], msem)
                        pltpu.async_copy(qsrc, q_v.at[pn], msem)
                with jax.named_scope("cmp"):
                    compute(b)

            def norm_body(lg, _):
                sl = pl.ds(lg * 16, 16)
                lanes = lg * 16 + iota
                inv = 1.0 / mden_v[1, sl]
                for c in range(CG):
                    rr = (jnp.full((16,), c, jnp.int32) - iota) & (CG - 1)
                    out2_v[c, sl] = plsc.load_gather(out_v,
                                                     [rr, lanes]) * inv
                return 0

            lax.fori_loop(0, nlg, norm_body, 0)
            with jax.named_scope("outw"):
                pltpu.sync_copy(
                    out2_v, out_hbm.at[pl.ds(g * CG, CG), pl.ds(px0, PX)])

            @pl.when(i < SPW - 1)
            def _():
                with jax.named_scope("mwait"):
                    isrc, wsrc, qsrc = meta_srcs(sup + 1)
                    pn = 1 - ps
                    pltpu.make_async_copy(isrc, idx_v.at[pn], msem).wait()
                    pltpu.make_async_copy(wsrc, wgt_v.at[pn], msem).wait()
                    pltpu.make_async_copy(qsrc, q_v.at[pn], msem).wait()

            return 0

        lax.fori_loop(0, SPW, sup_body, 0)

    return run(tab, idx, wgt, qt)


def _mlp_body(x_ref, w1_ref, b1_ref, w2_ref, b2_ref, o_ref):
    h = (jnp.dot(w1_ref[...], x_ref[...],
                 preferred_element_type=jnp.float32)
         + b1_ref[...][:, None])
    h = h * 0.5 * (1.0 + lax.erf(h * (2.0 ** -0.5)))
    o_ref[...] = (x_ref[...]
                  + jnp.dot(w2_ref[...], h,
                            preferred_element_type=jnp.float32)
                  + b2_ref[...][:, None])


def _run_mlp(x, W1, b1, W2, b2):
    return pl.pallas_call(
        _mlp_body,
        out_shape=jax.ShapeDtypeStruct((C, HW), jnp.float32),
    )(x, W1, b1, W2, b2)


def kernel(q, k, v, offset, Wq, bq, Wk, bk, Wv, bv, W1, b1, W2, b2):
    qf = q.reshape(C, HW)
    kf = k.reshape(CLIP, C, HW)
    vf = v.reshape(CLIP, C, HW)
    offr = offset.reshape(CLIP, GROUPS, ATTN, 2, SB, 128)
    qt, pw = _run_proj(qf, kf, vf, Wq, bq, Wk, bk, Wv, bv)
    wpm = pw.reshape(CLIP, GROUPS, CG, HW).transpose(0, 1, 3, 2)
    tab = jnp.concatenate(
        [wpm, jnp.roll(wpm, -1, axis=2), jnp.roll(wpm, -W, axis=2),
         jnp.roll(wpm, -W - 1, axis=2)],
        axis=3).reshape(CLIP * GROUPS * HW, 4 * CG)
    idx, wgt = _run_gen(offr)
    attn_out = _sc_attn(tab, idx, wgt, qt)
    out = _run_mlp(attn_out, W1, b1, W2, b2)
    return out.reshape(1, 1, C, H, W)


# speedup vs baseline: 1.9650x; 1.0075x over previous
import functools

import jax
import jax.numpy as jnp
from jax import lax
from jax.experimental import pallas as pl
from jax.experimental.pallas import tpu as pltpu
from jax.experimental.pallas import tpu_sc as plsc

C = 192
GROUPS = 12
CLIP = 2
H = 64
W = 64
HW = H * W
ATTN = 9
CG = C // GROUPS
NA = CLIP * ATTN
NJ = NA
NJP = 24
NWT = 4 * NA
SB = HW // 128
PX = 64
NSUP = GROUPS * (HW // PX)
NW = 32
SPW = NSUP // NW
AB = 3
JB = 2 * AB
NB = NA // AB
SCALE = float(CG) ** -0.5
NEG = -3.0e38


def _proj_body(qf_ref, kf_ref, vf_ref, wq_ref, bq_ref, wk_ref, bk_ref,
               wv_ref, bv_ref, qt_ref, pw_ref):
    t = pl.program_id(0)

    @pl.when(t == 0)
    def _():
        qt_ref[...] = (jnp.dot(wq_ref[...], qf_ref[...],
                               preferred_element_type=jnp.float32)
                       + bq_ref[...][:, None]) * SCALE

    kp = (jnp.dot(wk_ref[...], kf_ref[0],
                  preferred_element_type=jnp.float32)
          + bk_ref[...][:, None])
    vp = (jnp.dot(wv_ref[...], vf_ref[0],
                  preferred_element_type=jnp.float32)
          + bv_ref[...][:, None])
    kb = lax.bitcast_convert_type(kp.astype(jnp.bfloat16), jnp.uint16)
    vb = lax.bitcast_convert_type(vp.astype(jnp.bfloat16), jnp.uint16)
    pw_ref[0] = (vb.astype(jnp.int32) << 16) | kb.astype(jnp.int32)


def _run_proj(qf, kf, vf, Wq, bq, Wk, bk, Wv, bv):
    full = lambda *s: pl.BlockSpec(s, lambda t: (0,) * len(s))
    return pl.pallas_call(
        _proj_body,
        grid=(CLIP,),
        in_specs=[
            full(C, HW),
            pl.BlockSpec((1, C, HW), lambda t: (t, 0, 0)),
            pl.BlockSpec((1, C, HW), lambda t: (t, 0, 0)),
            full(C, C), full(C), full(C, C), full(C), full(C, C), full(C),
        ],
        out_specs=(
            full(C, HW),
            pl.BlockSpec((1, C, HW), lambda t: (t, 0, 0)),
        ),
        out_shape=(
            jax.ShapeDtypeStruct((C, HW), jnp.float32),
            jax.ShapeDtypeStruct((CLIP, C, HW), jnp.int32),
        ),
    )(qf, kf, vf, Wq, bq, Wk, bk, Wv, bv)


def _gen_body(off_ref, idx_ref, wgt_ref):
    g = pl.program_id(0)
    shp = (SB, 128)
    i0 = lax.broadcasted_iota(jnp.int32, shp, 0)
    i1 = lax.broadcasted_iota(jnp.int32, shp, 1)
    yi = 2 * i0 + (i1 >> 6)
    xi = i1 & 63
    yf = yi.astype(jnp.float32)
    xf = xi.astype(jnp.float32)
    for t in range(CLIP):
        base = (t * GROUPS + g) * HW
        for a in range(ATTN):
            dy = float(a // 3 - 1)
            dx = float(a % 3 - 1)
            sy = yf + dy + off_ref[t, 0, a, 0]
            sx = xf + dx + off_ref[t, 0, a, 1]
            y0 = jnp.floor(sy)
            x0 = jnp.floor(sx)
            wy = sy - y0
            wx = sx - x0
            vy0 = ((y0 >= 0.0) & (y0 <= H - 1.0)).astype(jnp.float32)
            vy1 = ((y0 >= -1.0) & (y0 <= H - 2.0)).astype(jnp.float32)
            vx0 = ((x0 >= 0.0) & (x0 <= W - 1.0)).astype(jnp.float32)
            vx1 = ((x0 >= -1.0) & (x0 <= W - 2.0)).astype(jnp.float32)
            yc0 = jnp.clip(y0, 0.0, H - 1.0).astype(jnp.int32)
            xc0 = jnp.clip(x0, 0.0, W - 1.0).astype(jnp.int32)
            negx = x0 < 0.0
            wh0 = jnp.where(negx, wx * vx1, (1.0 - wx) * vx0)
            wh1 = jnp.where(negx, 0.0, wx * vx1)
            negy = y0 < 0.0
            wv0 = jnp.where(negy, wy * vy1, (1.0 - wy) * vy0)
            wv1 = jnp.where(negy, 0.0, wy * vy1)
            aa = t * ATTN + a
            idx_ref[0, :, aa, :] = base + yc0 * W + xc0
            wgt_ref[0, :, 4 * aa + 0, :] = wv0 * wh0
            wgt_ref[0, :, 4 * aa + 1, :] = wv0 * wh1
            wgt_ref[0, :, 4 * aa + 2, :] = wv1 * wh0
            wgt_ref[0, :, 4 * aa + 3, :] = wv1 * wh1


def _run_gen(offr):
    return pl.pallas_call(
        _gen_body,
        grid=(GROUPS,),
        in_specs=[pl.BlockSpec((CLIP, 1, ATTN, 2, SB, 128),
                               lambda g: (0, g, 0, 0, 0, 0))],
        out_specs=(
            pl.BlockSpec((1, SB, NJP, 128), lambda g: (g, 0, 0, 0)),
            pl.BlockSpec((1, SB, NWT, 128), lambda g: (g, 0, 0, 0)),
        ),
        out_shape=(
            jax.ShapeDtypeStruct((GROUPS, SB, NJP, 128), jnp.int32),
            jax.ShapeDtypeStruct((GROUPS, SB, NWT, 128), jnp.float32),
        ),
    )(offr)


def _sc_attn(tab, idx, wgt, qt):
    mesh = plsc.VectorSubcoreMesh(core_axis_name="c", subcore_axis_name="s")
    hmask = jnp.int32(-65536)

    @functools.partial(
        pl.kernel,
        out_type=jax.ShapeDtypeStruct((C, HW), jnp.float32),
        mesh=mesh,
        compiler_params=pltpu.CompilerParams(use_tc_tiling_on_sc=False,
                                             needs_layout_passes=False),
        scratch_types=[
            pltpu.VMEM((2, NJ, PX), jnp.int32),
            pltpu.VMEM((2, NWT, PX), jnp.float32),
            pltpu.VMEM((2, CG, PX), jnp.float32),
            pltpu.VMEM((NJ + AB, PX, 4 * CG), jnp.int32),
            pltpu.VMEM((2, PX), jnp.float32),
            pltpu.VMEM((CG, PX), jnp.float32),
            pltpu.VMEM((CG, PX), jnp.float32),
            pltpu.SemaphoreType.DMA,
            pltpu.SemaphoreType.DMA,
            pltpu.SemaphoreType.DMA,
            pltpu.SemaphoreType.DMA,
            pltpu.SemaphoreType.DMA,
            pltpu.SemaphoreType.DMA,
            pltpu.SemaphoreType.DMA,
        ],
    )
    def run(tab_hbm, idx_hbm, wgt_hbm, qt_hbm, out_hbm,
            idx_v, wgt_v, q_v, rows_v, mden_v, out_v, out2_v,
            gsem0, gsem1, gsem2, gsem3, gsem4, gsem5, msem):
        wid = lax.axis_index("s") * 2 + lax.axis_index("c")
        iota = lax.iota(jnp.int32, 16)
        gsems = (gsem0, gsem1, gsem2, gsem3, gsem4, gsem5)
        nlg = PX // 16
        spg = HW // PX

        def meta_srcs(sup):
            g = sup // spg
            sb = (sup % spg) // 2
            hf = sup % 2
            px0 = (sup % spg) * PX
            return (idx_hbm.at[g, sb, pl.ds(0, NJ), pl.ds(hf * PX, PX)],
                    wgt_hbm.at[g, sb, :, pl.ds(hf * PX, PX)],
                    qt_hbm.at[pl.ds(g * CG, CG), pl.ds(px0, PX)])

        i0src, w0src, q0src = meta_srcs(wid * SPW)
        pltpu.sync_copy(i0src, idx_v.at[0])
        pltpu.sync_copy(w0src, wgt_v.at[0])
        pltpu.sync_copy(q0src, q_v.at[0])

        def issue_b0(ps_next):
            return [
                pltpu.async_copy(tab_hbm.at[idx_v.at[ps_next, jl]],
                                 rows_v.at[NJ + jl], gsems[0])
                for jl in range(AB)
            ]

        issue_b0(0)

        def sup_body(i, _):
            sup = wid * SPW + i
            g = sup // spg
            px0 = (sup % spg) * PX
            ps = lax.rem(i, 2)

            def issue(b):
                return [
                    pltpu.async_copy(tab_hbm.at[idx_v.at[ps, b * AB + jl]],
                                     rows_v.at[b * AB + jl],
                                     gsems[b])
                    for jl in range(AB)
                ]

            def init_body(lg, _):
                sl = pl.ds(lg * 16, 16)
                mden_v[0, sl] = jnp.full((16,), NEG, jnp.float32)
                mden_v[1, sl] = jnp.zeros((16,), jnp.float32)
                for c in range(CG):
                    out_v[c, sl] = jnp.zeros((16,), jnp.float32)
                return 0

            lax.fori_loop(0, nlg, init_body, 0)

            def compute(b):
                def a_body(al, _):
                    aa = b * AB + al

                    def lg_body(lg, _):
                        sl = pl.ds(lg * 16, 16)
                        lanes = lg * 16 + iota
                        ws = [wgt_v[ps, aa * 4 + tt, sl]
                              for tt in range(4)]
                        jv = jnp.full((16,), (NJ if b == 0 else b * AB)
                                      + al, jnp.int32)
                        psv = jnp.full((16,), ps, jnp.int32)
                        s = jnp.zeros((16,), jnp.float32)
                        vss = []
                        for c in range(CG):
                            crot = (jnp.full((16,), c, jnp.int32)
                                    + iota) & (CG - 1)
                            ks = jnp.zeros((16,), jnp.float32)
                            vs = jnp.zeros((16,), jnp.float32)
                            for tt in range(4):
                                word = plsc.load_gather(
                                    rows_v, [jv, lanes, crot + tt * CG])
                                kf = plsc.bitcast(word << 16,
                                                  jnp.float32)
                                vf = plsc.bitcast(word & hmask,
                                                  jnp.float32)
                                ks = ks + ws[tt] * kf
                                vs = vs + ws[tt] * vf
                            qv = plsc.load_gather(q_v, [psv, crot, lanes])
                            s = s + qv * ks
                            vss.append(vs)
                        m0 = mden_v[0, sl]
                        m1 = jnp.maximum(m0, s)
                        c1 = jnp.exp(m0 - m1)
                        e = jnp.exp(s - m1)
                        mden_v[0, sl] = m1
                        mden_v[1, sl] = mden_v[1, sl] * c1 + e
                        for c in range(CG):
                            out_v[c, sl] = out_v[c, sl] * c1 + e * vss[c]
                        return 0

                    lax.fori_loop(0, nlg, lg_body, 0)
                    return 0

                lax.fori_loop(0, AB, a_body, 0)

            with jax.named_scope("gissue"):
                pends = {b: issue(b) for b in range(1, NB)}
            for b in range(NB):
                with jax.named_scope("gwait"):
                    if b == 0:
                        for jl in range(AB):
                            pltpu.make_async_copy(
                                tab_hbm.at[idx_v.at[ps, jl]],
                                rows_v.at[NJ + jl], gsems[0]).wait()
                    else:
                        for cp in pends[b]:
                            cp.wait()
                if b == 0:
                    @pl.when(i < SPW - 1)
                    def _():
                        isrc, wsrc, qsrc = meta_srcs(sup + 1)
                        pn = 1 - ps
                        pltpu.async_copy(isrc, idx_v.at[pn], msem)
                        pltpu.async_copy(wsrc, wgt_v.at[pn], msem)
                        pltpu.async_copy(qsrc, q_v.at[pn], msem)
                with jax.named_scope("cmp"):
                    compute(b)

            def norm_body(lg, _):
                sl = pl.ds(lg * 16, 16)
                lanes = lg * 16 + iota
                inv = 1.0 / mden_v[1, sl]
                for c in range(CG):
                    rr = (jnp.full((16,), c, jnp.int32) - iota) & (CG - 1)
                    out2_v[c, sl] = plsc.load_gather(out_v,
                                                     [rr, lanes]) * inv
                return 0

            lax.fori_loop(0, nlg, norm_body, 0)
            with jax.named_scope("outw"):
                pltpu.sync_copy(
                    out2_v, out_hbm.at[pl.ds(g * CG, CG), pl.ds(px0, PX)])

            @pl.when(i < SPW - 1)
            def _():
                with jax.named_scope("mwait"):
                    isrc, wsrc, qsrc = meta_srcs(sup + 1)
                    pn = 1 - ps
                    pltpu.make_async_copy(isrc, idx_v.at[pn], msem).wait()
                    pltpu.make_async_copy(wsrc, wgt_v.at[pn], msem).wait()
                    pltpu.make_async_copy(qsrc, q_v.at[pn], msem).wait()
                issue_b0(pn)

            return 0

        lax.fori_loop(0, SPW, sup_body, 0)

    return run(tab, idx, wgt, qt)


def _mlp_body(x_ref, w1_ref, b1_ref, w2_ref, b2_ref, o_ref):
    h = (jnp.dot(w1_ref[...], x_ref[...],
                 preferred_element_type=jnp.float32)
         + b1_ref[...][:, None])
    h = h * 0.5 * (1.0 + lax.erf(h * (2.0 ** -0.5)))
    o_ref[...] = (x_ref[...]
                  + jnp.dot(w2_ref[...], h,
                            preferred_element_type=jnp.float32)
                  + b2_ref[...][:, None])


def _run_mlp(x, W1, b1, W2, b2):
    return pl.pallas_call(
        _mlp_body,
        out_shape=jax.ShapeDtypeStruct((C, HW), jnp.float32),
    )(x, W1, b1, W2, b2)


def kernel(q, k, v, offset, Wq, bq, Wk, bk, Wv, bv, W1, b1, W2, b2):
    qf = q.reshape(C, HW)
    kf = k.reshape(CLIP, C, HW)
    vf = v.reshape(CLIP, C, HW)
    offr = offset.reshape(CLIP, GROUPS, ATTN, 2, SB, 128)
    qt, pw = _run_proj(qf, kf, vf, Wq, bq, Wk, bk, Wv, bv)
    wpm = pw.reshape(CLIP, GROUPS, CG, HW).transpose(0, 1, 3, 2)
    tab = jnp.concatenate(
        [wpm, jnp.roll(wpm, -1, axis=2), jnp.roll(wpm, -W, axis=2),
         jnp.roll(wpm, -W - 1, axis=2)],
        axis=3).reshape(CLIP * GROUPS * HW, 4 * CG)
    idx, wgt = _run_gen(offr)
    attn_out = _sc_attn(tab, idx, wgt, qt)
    out = _run_mlp(attn_out, W1, b1, W2, b2)
    return out.reshape(1, 1, C, H, W)
